# baseline (speedup 1.0000x reference)
import dataclasses
import functools

import jax
import jax.numpy as jnp
from jax import lax
from jax.experimental import pallas as pl
from jax.experimental.pallas import tpu as pltpu
from jax.experimental.pallas import tpu_sc as plsc

D = 128
NC = 2
NS = 16
LANES = 16
CHUNK = 9504
ALLOC = 9600
TRASH = 9592
G = 112
W = 1024
FLUSH_AT = G - 2 * LANES


def _sc_segment_sum(x, src, dst, ew, n_nodes):
    e_pad = src.shape[0]
    ept = e_pad // NS
    n_win = ept // W
    n_chunks = -(-n_nodes // CHUNK)
    chunks_per_core = n_chunks // NC
    min_chunk = n_nodes - (n_chunks - 1) * CHUNK
    full_rows = (min_chunk // NS) // 8 * 8
    assert CHUNK - full_rows * NS <= NS * 8

    mesh = plsc.VectorSubcoreMesh(core_axis_name="c", subcore_axis_name="s")
    cp = pltpu.CompilerParams()
    if "needs_layout_passes" in pltpu.CompilerParams.__dataclass_fields__:
        cp = dataclasses.replace(cp, needs_layout_passes=False)

    @functools.partial(
        pl.kernel,
        out_type=jax.ShapeDtypeStruct((n_nodes, D), jnp.float32),
        mesh=mesh,
        compiler_params=cp,
        scratch_types=[
            pltpu.VMEM_SHARED((ALLOC, D), jnp.float32),
            pltpu.VMEM((2, W), jnp.int32),
            pltpu.VMEM((2, W), jnp.int32),
            pltpu.VMEM((2, W), jnp.float32),
            pltpu.VMEM((3, G), jnp.int32),
            pltpu.VMEM((3, G), jnp.int32),
            pltpu.VMEM((3, G), jnp.float32),
            pltpu.VMEM((3, G, D), jnp.float32),
            pltpu.VMEM((8, D), jnp.float32),
            pltpu.SemaphoreType.DMA((3,)),
            pltpu.SemaphoreType.DMA((2,)),
            pltpu.SemaphoreType.DMA,
            pltpu.SemaphoreType.DMA((3,)),
        ],
    )
    def seg_kernel(x_hbm, src_hbm, dst_hbm, ew_hbm, out_hbm,
                   acc, wsrc, wdst, wew, csrc, cdst, cew, rows, zbuf, gsem,
                   wsem, zsem, ssem):
        c = lax.axis_index("c")
        s = lax.axis_index("s")
        wid = s * NC + c

        zero16f = jnp.zeros((LANES,), jnp.float32)
        iota16 = jnp.arange(LANES, dtype=jnp.int32)
        zero16i = jnp.zeros((LANES,), jnp.int32)

        @plsc.parallel_loop(0, 8)
        def _(r):
            for l in range(D // LANES):
                zbuf[r, pl.ds(l * LANES, LANES)] = zero16f

        def refill(p):
            @plsc.parallel_loop(0, G, LANES)
            def _(i):
                csrc[p, pl.ds(i, LANES)] = wid * G + i + iota16
                cdst[p, pl.ds(i, LANES)] = TRASH + (iota16 & 7)
                cew[p, pl.ds(i, LANES)] = zero16f

        def fire_gather(p):
            pltpu.async_copy(x_hbm.at[csrc.at[p]], rows.at[p], gsem.at[p])

        def commit3(a, b):
            pltpu.make_async_copy(
                x_hbm.at[csrc.at[a]], rows.at[a], gsem.at[a]).wait()

            @plsc.parallel_loop(0, G, LANES, unroll=2)
            def _(rb):
                rb16 = pl.multiple_of(rb, LANES)
                ew16 = cew[a, pl.ds(rb16, LANES)]
                for k in range(LANES):
                    sval = ew16[k]
                    for l in range(D // LANES):
                        rows[a, rb16 + k, pl.ds(l * LANES, LANES)] = (
                            rows[a, rb16 + k, pl.ds(l * LANES, LANES)] * sval)

            pltpu.make_async_copy(
                rows.at[b], acc.at[cdst.at[b]], ssem.at[b]).wait()
            pltpu.async_copy(rows.at[a], acc.at[cdst.at[a]], ssem.at[a],
                             add=True)
            refill(b)

        @pl.loop(0, chunks_per_core)
        def _(phase):
            chunk = c * chunks_per_core + phase
            lo = chunk * CHUNK
            hi = jnp.minimum(lo + CHUNK, n_nodes)

            @pl.loop(0, ALLOC // NS // 8)
            def _(k):
                acc_off = pl.multiple_of(s * (ALLOC // NS) + k * 8, 8)
                pltpu.async_copy(zbuf, acc.at[pl.ds(acc_off, 8)], zsem)
            refill(0)
            refill(1)
            refill(2)

            @pl.loop(0, ALLOC // NS // 8)
            def _(k):
                acc_off = pl.multiple_of(s * (ALLOC // NS) + k * 8, 8)
                pltpu.make_async_copy(
                    zbuf, acc.at[pl.ds(acc_off, 8)], zsem).wait()
            plsc.subcore_barrier()
            fire_gather(2)
            pltpu.async_copy(rows.at[1], acc.at[cdst.at[1]], ssem.at[1],
                             add=True)

            def fire_win(w, b):
                base = pl.multiple_of(s * ept + w * W, W)
                pltpu.async_copy(src_hbm.at[pl.ds(base, W)], wsrc.at[b],
                                 wsem.at[b])
                pltpu.async_copy(dst_hbm.at[pl.ds(base, W)], wdst.at[b],
                                 wsem.at[b])
                pltpu.async_copy(ew_hbm.at[pl.ds(base, W)], wew.at[b],
                                 wsem.at[b])

            def wait_win(w, b):
                base = pl.multiple_of(s * ept + w * W, W)
                pltpu.make_async_copy(src_hbm.at[pl.ds(base, W)], wsrc.at[b],
                                      wsem.at[b]).wait()
                pltpu.make_async_copy(dst_hbm.at[pl.ds(base, W)], wdst.at[b],
                                      wsem.at[b]).wait()
                pltpu.make_async_copy(ew_hbm.at[pl.ds(base, W)], wew.at[b],
                                      wsem.at[b]).wait()

            def process_window(b, wcarry):
                def pair_body(j, carry):
                    cnt, par = carry
                    o1 = pl.multiple_of(j * 2 * LANES, LANES)
                    o2 = pl.multiple_of(j * 2 * LANES + LANES, LANES)
                    sv1 = wsrc[b, pl.ds(o1, LANES)]
                    dv1 = wdst[b, pl.ds(o1, LANES)]
                    wv1 = wew[b, pl.ds(o1, LANES)]
                    sv2 = wsrc[b, pl.ds(o2, LANES)]
                    dv2 = wdst[b, pl.ds(o2, LANES)]
                    wv2 = wew[b, pl.ds(o2, LANES)]
                    m1 = (dv1 >= lo) & (dv1 < hi)
                    m2 = (dv2 >= lo) & (dv2 < hi)
                    mi1 = m1.astype(jnp.int32)
                    mi2 = m2.astype(jnp.int32)
                    cs1 = plsc.cumsum(mi1)
                    cs2 = plsc.cumsum(mi2)
                    pos1 = cnt + cs1 - mi1
                    c1 = cnt + cs1[15]
                    pos2 = c1 + cs2 - mi2
                    c2 = c1 + cs2[15]
                    ps = zero16i + par
                    plsc.store_scatter(csrc, [ps, pos1], sv1, mask=m1)
                    plsc.store_scatter(cdst, [ps, pos1], dv1 - lo, mask=m1)
                    plsc.store_scatter(cew, [ps, pos1], wv1, mask=m1)
                    plsc.store_scatter(csrc, [ps, pos2], sv2, mask=m2)
                    plsc.store_scatter(cdst, [ps, pos2], dv2 - lo, mask=m2)
                    plsc.store_scatter(cew, [ps, pos2], wv2, mask=m2)
                    do_flush = c2 >= FLUSH_AT

                    @pl.when(do_flush & (par == 0))
                    def _():
                        fire_gather(0)
                        commit3(2, 1)

                    @pl.when(do_flush & (par == 1))
                    def _():
                        fire_gather(1)
                        commit3(0, 2)

                    @pl.when(do_flush & (par == 2))
                    def _():
                        fire_gather(2)
                        commit3(1, 0)

                    npar = jnp.where(par == 2, 0, par + 1)
                    return (jnp.where(do_flush, 0, c2),
                            jnp.where(do_flush, npar, par))

                return lax.fori_loop(0, W // (2 * LANES), pair_body, wcarry)

            fire_win(0, 0)

            def win2_body(t, wcarry):
                w0 = t * 2
                fire_win(w0 + 1, 1)
                wait_win(w0, 0)
                wcarry = process_window(0, wcarry)

                @pl.when(w0 + 2 < n_win)
                def _():
                    fire_win(w0 + 2, 0)

                wait_win(w0 + 1, 1)
                return process_window(1, wcarry)

            count, parity = lax.fori_loop(
                0, n_win // 2, win2_body, (jnp.int32(0), jnp.int32(0)))

            def drain_body(_, par):
                prev = lax.rem(par + 2, 3)
                prev2 = lax.rem(par + 1, 3)
                pltpu.async_copy(x_hbm.at[csrc.at[par]], rows.at[par],
                                 gsem.at[par])
                pltpu.make_async_copy(
                    x_hbm.at[csrc.at[prev]], rows.at[prev],
                    gsem.at[prev]).wait()

                @plsc.parallel_loop(0, G, LANES)
                def _(rb):
                    rb16 = pl.multiple_of(rb, LANES)
                    ew16 = cew[prev, pl.ds(rb16, LANES)]
                    for k in range(LANES):
                        sval = ew16[k]
                        for l in range(D // LANES):
                            rows[prev, rb16 + k, pl.ds(l * LANES, LANES)] = (
                                rows[prev, rb16 + k,
                                     pl.ds(l * LANES, LANES)] * sval)

                pltpu.make_async_copy(
                    rows.at[prev2], acc.at[cdst.at[prev2]],
                    ssem.at[prev2]).wait()
                pltpu.async_copy(rows.at[prev], acc.at[cdst.at[prev]],
                                 ssem.at[prev], add=True)

                @plsc.parallel_loop(0, G, LANES)
                def _(i):
                    csrc[prev2, pl.ds(i, LANES)] = wid * G + i + iota16
                    cdst[prev2, pl.ds(i, LANES)] = TRASH + (iota16 & 7)
                    cew[prev2, pl.ds(i, LANES)] = zero16f

                return lax.rem(par + 1, 3)

            par_end = lax.fori_loop(0, 4, drain_body, parity)
            qg = lax.rem(par_end + 2, 3)
            qs = lax.rem(par_end + 1, 3)
            pltpu.make_async_copy(
                x_hbm.at[csrc.at[qg]], rows.at[qg], gsem.at[qg]).wait()
            pltpu.make_async_copy(
                rows.at[qs], acc.at[cdst.at[qs]], ssem.at[qs]).wait()
            plsc.subcore_barrier()

            rc = hi - lo
            row0 = pl.multiple_of(s * full_rows, 8)
            pltpu.sync_copy(acc.at[pl.ds(row0, full_rows)],
                            out_hbm.at[pl.ds(lo + row0, full_rows)])
            tail_base = full_rows * NS

            @pl.when(tail_base + s * 8 < rc)
            def _():
                t0 = pl.multiple_of(tail_base + s * 8, 8)
                pltpu.sync_copy(acc.at[pl.ds(t0, 8)],
                                out_hbm.at[pl.ds(lo + t0, 8)])

            plsc.subcore_barrier()

    return seg_kernel(x, src, dst, ew)


def _tc_root(x, w_root_t, b2d):
    n = x.shape[0]
    blk = 1000
    grid = (n // blk,)

    def body(x_ref, wq_ref, b_ref, o_ref):
        o_ref[...] = jnp.dot(
            x_ref[...], wq_ref[...],
            preferred_element_type=jnp.float32) + b_ref[...]

    return pl.pallas_call(
        body,
        grid=grid,
        in_specs=[
            pl.BlockSpec((blk, D), lambda i: (i, 0)),
            pl.BlockSpec((D, D), lambda i: (0, 0)),
            pl.BlockSpec((1, D), lambda i: (0, 0)),
        ],
        out_specs=pl.BlockSpec((blk, D), lambda i: (i, 0)),
        out_shape=jax.ShapeDtypeStruct((n, D), jnp.float32),
    )(x, w_root_t, b2d)


def _tc_rel(aggr, w_rel_t, root):
    n = aggr.shape[0]
    blk = 1000
    grid = (n // blk,)

    def body(a_ref, wr_ref, r_ref, o_ref):
        o_ref[...] = jnp.dot(
            a_ref[...], wr_ref[...],
            preferred_element_type=jnp.float32) + r_ref[...]

    return pl.pallas_call(
        body,
        grid=grid,
        in_specs=[
            pl.BlockSpec((blk, D), lambda i: (i, 0)),
            pl.BlockSpec((D, D), lambda i: (0, 0)),
            pl.BlockSpec((blk, D), lambda i: (i, 0)),
        ],
        out_specs=pl.BlockSpec((blk, D), lambda i: (i, 0)),
        out_shape=jax.ShapeDtypeStruct((n, D), jnp.float32),
    )(aggr, w_rel_t, root)


def kernel(x, edge_index, edge_weights, W_rel, b_rel, W_root):
    n_nodes = x.shape[0]
    n_elec = 19
    repeat = n_nodes // n_elec
    ew_full = jnp.tile(edge_weights, repeat)

    src = edge_index[0]
    dst = edge_index[1]
    e = src.shape[0]
    e_pad = -(-e // (NS * W)) * (NS * W)
    pad = e_pad - e
    src_p = jnp.concatenate([src, jnp.zeros((pad,), jnp.int32)])
    dst_p = jnp.concatenate([dst, jnp.full((pad,), -1, jnp.int32)])
    ew_p = jnp.concatenate([ew_full, jnp.zeros((pad,), jnp.float32)])

    root = _tc_root(x, W_root.T, b_rel[None, :])
    aggr = _sc_segment_sum(x, src_p, dst_p, ew_p, n_nodes)
    return _tc_rel(aggr, W_rel.T, root)


# speedup vs baseline: 1.0677x; 1.0677x over previous
import dataclasses
import functools

import jax
import jax.numpy as jnp
from jax import lax
from jax.experimental import pallas as pl
from jax.experimental.pallas import tpu as pltpu
from jax.experimental.pallas import tpu_sc as plsc

D = 128
NC = 2
NS = 16
LANES = 16
CHUNK = 9504
ALLOC = 9600
TRASH = 9592
G = 128
W = 2048
FLUSH_AT = G - 2 * LANES


def _sc_segment_sum(x, src, dst, ew, n_nodes):
    e_pad = src.shape[0]
    ept = e_pad // NS
    n_win = ept // W
    n_chunks = -(-n_nodes // CHUNK)
    chunks_per_core = n_chunks // NC
    min_chunk = n_nodes - (n_chunks - 1) * CHUNK
    full_rows = (min_chunk // NS) // 8 * 8
    assert CHUNK - full_rows * NS <= NS * 8

    mesh = plsc.VectorSubcoreMesh(core_axis_name="c", subcore_axis_name="s")
    cp = pltpu.CompilerParams()
    if "needs_layout_passes" in pltpu.CompilerParams.__dataclass_fields__:
        cp = dataclasses.replace(cp, needs_layout_passes=False)

    @functools.partial(
        pl.kernel,
        out_type=jax.ShapeDtypeStruct((n_nodes, D), jnp.float32),
        mesh=mesh,
        compiler_params=cp,
        scratch_types=[
            pltpu.VMEM_SHARED((ALLOC, D), jnp.float32),
            pltpu.VMEM((2, W), jnp.int32),
            pltpu.VMEM((2, W), jnp.int32),
            pltpu.VMEM((2, W), jnp.float32),
            pltpu.VMEM((2, G), jnp.int32),
            pltpu.VMEM((2, G), jnp.int32),
            pltpu.VMEM((2, G), jnp.float32),
            pltpu.VMEM((2, G, D), jnp.float32),
            pltpu.VMEM((40, D), jnp.float32),
            pltpu.SemaphoreType.DMA((2,)),
            pltpu.SemaphoreType.DMA((2,)),
            pltpu.SemaphoreType.DMA,
        ],
    )
    def seg_kernel(x_hbm, src_hbm, dst_hbm, ew_hbm, out_hbm,
                   acc, wsrc, wdst, wew, csrc, cdst, cew, rows, zbuf, gsem,
                   wsem, zsem):
        c = lax.axis_index("c")
        s = lax.axis_index("s")
        wid = s * NC + c

        zero16f = jnp.zeros((LANES,), jnp.float32)
        iota16 = jnp.arange(LANES, dtype=jnp.int32)
        zero16i = jnp.zeros((LANES,), jnp.int32)

        @plsc.parallel_loop(0, 40)
        def _(r):
            for l in range(D // LANES):
                zbuf[r, pl.ds(l * LANES, LANES)] = zero16f

        def refill(p):
            @plsc.parallel_loop(0, G, LANES)
            def _(i):
                csrc[p, pl.ds(i, LANES)] = wid * G + i + iota16
                cdst[p, pl.ds(i, LANES)] = TRASH + (iota16 & 7)
                cew[p, pl.ds(i, LANES)] = zero16f

        def fire_gather(p):
            pltpu.async_copy(x_hbm.at[csrc.at[p]], rows.at[p], gsem.at[p])

        def commit(p):
            pltpu.make_async_copy(
                x_hbm.at[csrc.at[p]], rows.at[p], gsem.at[p]).wait()

            @plsc.parallel_loop(0, G, LANES, unroll=2)
            def _(rb):
                rb16 = pl.multiple_of(rb, LANES)
                ew16 = cew[p, pl.ds(rb16, LANES)]
                for k in range(LANES):
                    sval = ew16[k]
                    for l in range(D // LANES):
                        rows[p, rb16 + k, pl.ds(l * LANES, LANES)] = (
                            rows[p, rb16 + k, pl.ds(l * LANES, LANES)] * sval)

            pltpu.sync_copy(rows.at[p], acc.at[cdst.at[p]], add=True)
            refill(p)

        @pl.loop(0, chunks_per_core)
        def _(phase):
            chunk = c * chunks_per_core + phase
            lo = chunk * CHUNK
            hi = jnp.minimum(lo + CHUNK, n_nodes)

            @pl.loop(0, ALLOC // NS // 40)
            def _(k):
                acc_off = pl.multiple_of(s * (ALLOC // NS) + k * 40, 8)
                pltpu.async_copy(zbuf, acc.at[pl.ds(acc_off, 40)], zsem)
            refill(0)
            refill(1)

            @pl.loop(0, ALLOC // NS // 40)
            def _(k):
                acc_off = pl.multiple_of(s * (ALLOC // NS) + k * 40, 8)
                pltpu.make_async_copy(
                    zbuf, acc.at[pl.ds(acc_off, 40)], zsem).wait()
            plsc.subcore_barrier()
            fire_gather(1)

            def fire_win(w, b):
                base = pl.multiple_of(s * ept + w * W, W)
                pltpu.async_copy(src_hbm.at[pl.ds(base, W)], wsrc.at[b],
                                 wsem.at[b])
                pltpu.async_copy(dst_hbm.at[pl.ds(base, W)], wdst.at[b],
                                 wsem.at[b])
                pltpu.async_copy(ew_hbm.at[pl.ds(base, W)], wew.at[b],
                                 wsem.at[b])

            def wait_win(w, b):
                base = pl.multiple_of(s * ept + w * W, W)
                pltpu.make_async_copy(src_hbm.at[pl.ds(base, W)], wsrc.at[b],
                                      wsem.at[b]).wait()
                pltpu.make_async_copy(dst_hbm.at[pl.ds(base, W)], wdst.at[b],
                                      wsem.at[b]).wait()
                pltpu.make_async_copy(ew_hbm.at[pl.ds(base, W)], wew.at[b],
                                      wsem.at[b]).wait()

            def process_window(b, wcarry):
                def pair_body(j, carry):
                    cnt, par = carry
                    o1 = pl.multiple_of(j * 2 * LANES, LANES)
                    o2 = pl.multiple_of(j * 2 * LANES + LANES, LANES)
                    sv1 = wsrc[b, pl.ds(o1, LANES)]
                    dv1 = wdst[b, pl.ds(o1, LANES)]
                    wv1 = wew[b, pl.ds(o1, LANES)]
                    sv2 = wsrc[b, pl.ds(o2, LANES)]
                    dv2 = wdst[b, pl.ds(o2, LANES)]
                    wv2 = wew[b, pl.ds(o2, LANES)]
                    m1 = (dv1 >= lo) & (dv1 < hi)
                    m2 = (dv2 >= lo) & (dv2 < hi)
                    mi1 = m1.astype(jnp.int32)
                    mi2 = m2.astype(jnp.int32)
                    cs1 = plsc.cumsum(mi1)
                    cs2 = plsc.cumsum(mi2)
                    pos1 = cnt + cs1 - mi1
                    c1 = cnt + cs1[15]
                    pos2 = c1 + cs2 - mi2
                    c2 = c1 + cs2[15]
                    ps = zero16i + par
                    plsc.store_scatter(csrc, [ps, pos1], sv1, mask=m1)
                    plsc.store_scatter(cdst, [ps, pos1], dv1 - lo, mask=m1)
                    plsc.store_scatter(cew, [ps, pos1], wv1, mask=m1)
                    plsc.store_scatter(csrc, [ps, pos2], sv2, mask=m2)
                    plsc.store_scatter(cdst, [ps, pos2], dv2 - lo, mask=m2)
                    plsc.store_scatter(cew, [ps, pos2], wv2, mask=m2)
                    do_flush = c2 >= FLUSH_AT

                    @pl.when(do_flush & (par == 0))
                    def _():
                        fire_gather(0)
                        commit(1)

                    @pl.when(do_flush & (par == 1))
                    def _():
                        fire_gather(1)
                        commit(0)

                    return (jnp.where(do_flush, 0, c2),
                            jnp.where(do_flush, 1 - par, par))

                return lax.fori_loop(0, W // (2 * LANES), pair_body, wcarry)

            fire_win(0, 0)

            def win2_body(t, wcarry):
                w0 = t * 2
                fire_win(w0 + 1, 1)
                wait_win(w0, 0)
                wcarry = process_window(0, wcarry)

                @pl.when(w0 + 2 < n_win)
                def _():
                    fire_win(w0 + 2, 0)

                wait_win(w0 + 1, 1)
                return process_window(1, wcarry)

            count, parity = lax.fori_loop(
                0, n_win // 2, win2_body, (jnp.int32(0), jnp.int32(0)))

            def drain_body(_, par):
                pltpu.async_copy(x_hbm.at[csrc.at[par]], rows.at[par],
                                 gsem.at[par])
                q = 1 - par
                pltpu.make_async_copy(
                    x_hbm.at[csrc.at[q]], rows.at[q], gsem.at[q]).wait()

                @plsc.parallel_loop(0, G, LANES)
                def _(rb):
                    rb16 = pl.multiple_of(rb, LANES)
                    ew16 = cew[q, pl.ds(rb16, LANES)]
                    for k in range(LANES):
                        sval = ew16[k]
                        for l in range(D // LANES):
                            rows[q, rb16 + k, pl.ds(l * LANES, LANES)] = (
                                rows[q, rb16 + k, pl.ds(l * LANES, LANES)]
                                * sval)

                pltpu.sync_copy(rows.at[q], acc.at[cdst.at[q]], add=True)

                @plsc.parallel_loop(0, G, LANES)
                def _(i):
                    csrc[q, pl.ds(i, LANES)] = wid * G + i + iota16
                    cdst[q, pl.ds(i, LANES)] = TRASH + (iota16 & 7)
                    cew[q, pl.ds(i, LANES)] = zero16f

                return q

            par_end = lax.fori_loop(0, 2, drain_body, parity)
            q = 1 - par_end
            pltpu.make_async_copy(
                x_hbm.at[csrc.at[q]], rows.at[q], gsem.at[q]).wait()
            plsc.subcore_barrier()

            rc = hi - lo
            row0 = pl.multiple_of(s * full_rows, 8)
            pltpu.sync_copy(acc.at[pl.ds(row0, full_rows)],
                            out_hbm.at[pl.ds(lo + row0, full_rows)])
            tail_base = full_rows * NS

            @pl.when(tail_base + s * 8 < rc)
            def _():
                t0 = pl.multiple_of(tail_base + s * 8, 8)
                pltpu.sync_copy(acc.at[pl.ds(t0, 8)],
                                out_hbm.at[pl.ds(lo + t0, 8)])

            plsc.subcore_barrier()

    return seg_kernel(x, src, dst, ew)


def _tc_root(x, w_root_t, b2d):
    n = x.shape[0]
    blk = 1000
    grid = (n // blk,)

    def body(x_ref, wq_ref, b_ref, o_ref):
        o_ref[...] = jnp.dot(
            x_ref[...], wq_ref[...],
            preferred_element_type=jnp.float32) + b_ref[...]

    return pl.pallas_call(
        body,
        grid=grid,
        in_specs=[
            pl.BlockSpec((blk, D), lambda i: (i, 0)),
            pl.BlockSpec((D, D), lambda i: (0, 0)),
            pl.BlockSpec((1, D), lambda i: (0, 0)),
        ],
        out_specs=pl.BlockSpec((blk, D), lambda i: (i, 0)),
        out_shape=jax.ShapeDtypeStruct((n, D), jnp.float32),
    )(x, w_root_t, b2d)


def _tc_rel(aggr, w_rel_t, root):
    n = aggr.shape[0]
    blk = 1000
    grid = (n // blk,)

    def body(a_ref, wr_ref, r_ref, o_ref):
        o_ref[...] = jnp.dot(
            a_ref[...], wr_ref[...],
            preferred_element_type=jnp.float32) + r_ref[...]

    return pl.pallas_call(
        body,
        grid=grid,
        in_specs=[
            pl.BlockSpec((blk, D), lambda i: (i, 0)),
            pl.BlockSpec((D, D), lambda i: (0, 0)),
            pl.BlockSpec((blk, D), lambda i: (i, 0)),
        ],
        out_specs=pl.BlockSpec((blk, D), lambda i: (i, 0)),
        out_shape=jax.ShapeDtypeStruct((n, D), jnp.float32),
    )(aggr, w_rel_t, root)


def kernel(x, edge_index, edge_weights, W_rel, b_rel, W_root):
    n_nodes = x.shape[0]
    n_elec = 19
    repeat = n_nodes // n_elec
    ew_full = jnp.tile(edge_weights, repeat)

    src = edge_index[0]
    dst = edge_index[1]
    e = src.shape[0]
    e_pad = -(-e // (NS * W)) * (NS * W)
    pad = e_pad - e
    src_p = jnp.concatenate([src, jnp.zeros((pad,), jnp.int32)])
    dst_p = jnp.concatenate([dst, jnp.full((pad,), -1, jnp.int32)])
    ew_p = jnp.concatenate([ew_full, jnp.zeros((pad,), jnp.float32)])

    root = _tc_root(x, W_root.T, b_rel[None, :])
    aggr = _sc_segment_sum(x, src_p, dst_p, ew_p, n_nodes)
    return _tc_rel(aggr, W_rel.T, root)
